# baseline (speedup 1.0000x reference)
import functools

import jax
import jax.numpy as jnp
from jax.experimental import pallas as pl
from jax.experimental.pallas import tpu as pltpu

_VMEM_LIMIT = 60 * 1024 * 1024


def _stca_body(x1_ref, x2_ref, ws1_ref, ws2_ref, wt1_ref, wt2_ref,
               awb_ref, o_ref, *, inv_s):
    c = x1_ref.shape[2]
    a1 = jnp.sum(x1_ref[0], axis=0, keepdims=True) * inv_s
    a2 = jnp.sum(x2_ref[0], axis=0, keepdims=True) * inv_s
    hp = jax.lax.Precision.HIGHEST
    dn = (((1,), (1,)), ((), ()))
    h1 = jax.lax.dot_general(a1, ws1_ref[...], dn, precision=hp,
                             preferred_element_type=jnp.float32)
    s_out = jax.lax.dot_general(h1, ws2_ref[...], dn, precision=hp,
                                preferred_element_type=jnp.float32)
    h2 = jax.lax.dot_general(a2, wt1_ref[...], dn, precision=hp,
                             preferred_element_type=jnp.float32)
    t_out = jax.lax.dot_general(h2, wt2_ref[...], dn, precision=hp,
                                preferred_element_type=jnp.float32)
    c0 = awb_ref[0, 0] - awb_ref[1, 0]
    c1 = awb_ref[0, 1] - awb_ref[1, 1]
    cb = awb_ref[0, 2] - awb_ref[1, 2]
    d = s_out * c0 + t_out * c1 + cb
    lane = jax.lax.broadcasted_iota(jnp.int32, (1, 2 * c), 1)
    dd = jnp.repeat(d, 2, axis=1)
    sgn = 1.0 - 2.0 * (lane % 2).astype(jnp.float32)
    o_ref[0] = jax.nn.sigmoid(dd * sgn)


def kernel(x1, x2, ws1, ws2, wt1, wt2, aw, ab):
    N, C, T, W, H = x1.shape
    S = T * W * H
    x1t = x1.transpose(0, 3, 4, 2, 1).reshape(N, S, C)
    x2t = x2.transpose(0, 3, 4, 2, 1).reshape(N, S, C)
    awb = jnp.concatenate([aw, ab.reshape(2, 1)], axis=1)

    wspec = lambda shape: pl.BlockSpec(shape, lambda n: (0, 0))
    p = pl.pallas_call(
        functools.partial(_stca_body, inv_s=1.0 / S),
        out_shape=jax.ShapeDtypeStruct((N, 1, 2 * C), jnp.float32),
        grid=(N,),
        in_specs=[
            pl.BlockSpec((1, S, C), lambda n: (n, 0, 0)),
            pl.BlockSpec((1, S, C), lambda n: (n, 0, 0)),
            wspec(ws1.shape),
            wspec(ws2.shape),
            wspec(wt1.shape),
            wspec(wt2.shape),
            pl.BlockSpec(memory_space=pltpu.SMEM),
        ],
        out_specs=pl.BlockSpec((1, 1, 2 * C), lambda n: (n, 0, 0)),
        compiler_params=pltpu.CompilerParams(
            dimension_semantics=("parallel",),
            vmem_limit_bytes=_VMEM_LIMIT,
        ),
        cost_estimate=pl.CostEstimate(
            flops=int(2 * N * C * S),
            transcendentals=int(2 * N * C),
            bytes_accessed=int(2 * N * C * S * 4 + 2 * N * C * 4),
        ),
    )(x1t, x2t, ws1, ws2, wt1, wt2, awb)

    return p.reshape(N, C, 2, 1, 1, 1)
